# baseline (speedup 1.0000x reference)
import jax
import jax.numpy as jnp
from jax.experimental import pallas as pl


_SEQ_BLOCK = 512


def _add_kernel(x_ref, pos_ref, out_ref):
    out_ref[...] = x_ref[...] + pos_ref[...][None, :, :]


def kernel(x, pos_table):
    batch, seq_len, d_model = x.shape
    blk = _SEQ_BLOCK
    if seq_len % blk != 0:
        blk = seq_len
    grid = (seq_len // blk,)
    return pl.pallas_call(
        _add_kernel,
        grid=grid,
        in_specs=[
            pl.BlockSpec((batch, blk, d_model), lambda i: (0, i, 0)),
            pl.BlockSpec((blk, d_model), lambda i: (i, 0)),
        ],
        out_specs=pl.BlockSpec((batch, blk, d_model), lambda i: (0, i, 0)),
        out_shape=jax.ShapeDtypeStruct((batch, seq_len, d_model), x.dtype),
    )(x, pos_table[:seq_len])
